# baseline (device time: 10409 ns/iter reference)
import jax
import jax.numpy as jnp
from jax import lax
from jax.experimental import pallas as pl
from jax.experimental.pallas import tpu as pltpu

N_DEV = 4
B = 4


def _unpack_col(packed, rows):
    rt = rows // 128
    row_i = lax.broadcasted_iota(jnp.int32, (rows, 128), 0)
    lane_i = lax.broadcasted_iota(jnp.int32, (rows, 128), 1)
    tmp = jnp.broadcast_to(packed[0:1, :], (rows, 128))
    for t in range(1, rt):
        tmp = jnp.where(
            row_i >= t * 128,
            jnp.broadcast_to(packed[t:t + 1, :], (rows, 128)),
            tmp,
        )
    sel = jnp.where(lane_i == row_i % 128, tmp, 0.0)
    return jnp.sum(sel, axis=1, keepdims=True)


def kernel(x):
    m_per, n_per = x.shape
    rb = m_per // B
    rt = rb // 128

    def body(x_ref, out_ref, comm_ref, send_sems, recv_sems):
        my = lax.axis_index("i")
        wait_order = (1, 3, 2)

        barrier_sem = pltpu.get_barrier_semaphore()
        for d in range(1, N_DEV):
            pl.semaphore_signal(
                barrier_sem, inc=1,
                device_id=((my + d) % N_DEV,),
                device_id_type=pl.DeviceIdType.MESH,
            )

        rdmas = [None] * (B * N_DEV)
        es = [None] * B

        def merge_and_store(b):
            m_pk = comm_ref[b, 0, 0:rt, :]
            M = m_pk
            S = comm_ref[b, 0, rt:2 * rt, :]
            for d in wait_order:
                rdmas[b * N_DEV + d].wait_recv()
                m_j = comm_ref[b, d, 0:rt, :]
                s_j = comm_ref[b, d, rt:2 * rt, :]
                newM = jnp.maximum(M, m_j)
                S = S * jnp.exp(M - newM) + s_j * jnp.exp(m_j - newM)
                M = newM
            scale = _unpack_col(jnp.exp(m_pk - M) / S, rb)
            out_ref[b * rb:(b + 1) * rb, :] = es[b] * scale
            es[b] = None

        for b in range(B):
            xb = x_ref[b * rb:(b + 1) * rb, :]
            m = jnp.max(xb, axis=1, keepdims=True)
            e = jnp.exp(xb - m)
            s = jnp.sum(e, axis=1, keepdims=True)
            comm_ref[b, 0, 0:rt, :] = jnp.reshape(m, (rt, 128))
            comm_ref[b, 0, rt:2 * rt, :] = jnp.reshape(s, (rt, 128))
            if b == 0:
                pl.semaphore_wait(barrier_sem, N_DEV - 1)
            for d in range(1, N_DEV):
                rdma = pltpu.make_async_remote_copy(
                    src_ref=comm_ref.at[b, 0],
                    dst_ref=comm_ref.at[b, d],
                    send_sem=send_sems.at[b, d - 1],
                    recv_sem=recv_sems.at[b, d - 1],
                    device_id=((my + d) % N_DEV,),
                    device_id_type=pl.DeviceIdType.MESH,
                )
                rdma.start()
                rdmas[b * N_DEV + d] = rdma
            es[b] = e
            if b > 0:
                merge_and_store(b - 1)
        merge_and_store(B - 1)

        for b in range(B):
            for d in range(1, N_DEV):
                rdmas[b * N_DEV + d].wait_send()

    return pl.pallas_call(
        body,
        out_shape=jax.ShapeDtypeStruct((m_per, n_per), x.dtype),
        in_specs=[pl.BlockSpec(memory_space=pltpu.VMEM)],
        out_specs=pl.BlockSpec(memory_space=pltpu.VMEM),
        scratch_shapes=[
            pltpu.VMEM((B, N_DEV, 2 * rt, 128), jnp.float32),
            pltpu.SemaphoreType.DMA((B, N_DEV - 1)),
            pltpu.SemaphoreType.DMA((B, N_DEV - 1)),
        ],
        compiler_params=pltpu.CompilerParams(collective_id=0),
    )(x)


# device time: 7221 ns/iter; 1.4415x vs baseline; 1.4415x over previous
import jax
import jax.numpy as jnp
from jax import lax
from jax.experimental import pallas as pl
from jax.experimental.pallas import tpu as pltpu

N_DEV = 4
B = 2


def _unpack_col(packed, rows):
    rt = rows // 128
    row_i = lax.broadcasted_iota(jnp.int32, (rows, 128), 0)
    lane_i = lax.broadcasted_iota(jnp.int32, (rows, 128), 1)
    tmp = jnp.broadcast_to(packed[0:1, :], (rows, 128))
    for t in range(1, rt):
        tmp = jnp.where(
            row_i >= t * 128,
            jnp.broadcast_to(packed[t:t + 1, :], (rows, 128)),
            tmp,
        )
    sel = jnp.where(lane_i == row_i % 128, tmp, 0.0)
    return jnp.sum(sel, axis=1, keepdims=True)


def kernel(x):
    m_per, n_per = x.shape
    rb = m_per // B
    rt = rb // 128

    def body(x_ref, out_ref, comm_ref, send_sems, recv_sems):
        my = lax.axis_index("i")
        wait_order = (1, 3, 2)

        barrier_sem = pltpu.get_barrier_semaphore()
        for d in range(1, N_DEV):
            pl.semaphore_signal(
                barrier_sem, inc=1,
                device_id=((my + d) % N_DEV,),
                device_id_type=pl.DeviceIdType.MESH,
            )

        rdmas = [None] * (B * N_DEV)

        for b in range(B):
            xb = x_ref[b * rb:(b + 1) * rb, :]
            m = jnp.max(xb, axis=1, keepdims=True)
            e = jnp.exp(xb - m)
            s = jnp.sum(e, axis=1, keepdims=True)
            comm_ref[b, 0, 0:rt, :] = jnp.reshape(m, (rt, 128))
            comm_ref[b, 0, rt:2 * rt, :] = jnp.reshape(s, (rt, 128))
            if b == 0:
                pl.semaphore_wait(barrier_sem, N_DEV - 1)
            for d in range(1, N_DEV):
                rdma = pltpu.make_async_remote_copy(
                    src_ref=comm_ref.at[b, 0],
                    dst_ref=comm_ref.at[b, d],
                    send_sem=send_sems.at[b, d - 1],
                    recv_sem=recv_sems.at[b, d - 1],
                    device_id=((my + d) % N_DEV,),
                    device_id_type=pl.DeviceIdType.MESH,
                )
                rdma.start()
                rdmas[b * N_DEV + d] = rdma
            out_ref[b * rb:(b + 1) * rb, :] = e

        for b in range(B):
            m_pk = comm_ref[b, 0, 0:rt, :]
            M = m_pk
            S = comm_ref[b, 0, rt:2 * rt, :]
            for d in wait_order:
                rdmas[b * N_DEV + d].wait_recv()
                m_j = comm_ref[b, d, 0:rt, :]
                s_j = comm_ref[b, d, rt:2 * rt, :]
                newM = jnp.maximum(M, m_j)
                S = S * jnp.exp(M - newM) + s_j * jnp.exp(m_j - newM)
                M = newM
            scale = _unpack_col(jnp.exp(m_pk - M) / S, rb)
            out_ref[b * rb:(b + 1) * rb, :] = (
                out_ref[b * rb:(b + 1) * rb, :] * scale
            )

        for b in range(B):
            for d in range(1, N_DEV):
                rdmas[b * N_DEV + d].wait_send()

    return pl.pallas_call(
        body,
        out_shape=jax.ShapeDtypeStruct((m_per, n_per), x.dtype),
        in_specs=[pl.BlockSpec(memory_space=pltpu.VMEM)],
        out_specs=pl.BlockSpec(memory_space=pltpu.VMEM),
        scratch_shapes=[
            pltpu.VMEM((B, N_DEV, 2 * rt, 128), jnp.float32),
            pltpu.SemaphoreType.DMA((B, N_DEV - 1)),
            pltpu.SemaphoreType.DMA((B, N_DEV - 1)),
        ],
        compiler_params=pltpu.CompilerParams(collective_id=0),
    )(x)
